# baseline (speedup 1.0000x reference)
import functools

import jax
import jax.numpy as jnp
from jax import lax
from jax.experimental import pallas as pl
from jax.experimental.pallas import tpu as pltpu
from jax.experimental.pallas import tpu_sc as plsc


@functools.lru_cache(maxsize=None)
def _build(batch: int, hist: int, vocab: int, embed: int):
    info = plsc.get_sparse_core_info()
    nc, ns, nl = info.num_cores, info.num_subcores, info.num_lanes
    nw = nc * ns
    assert batch % (16 * nw) == 0 and hist % 8 == 0 and embed == 64
    n_hblk = hist // 8
    n_bblk = batch // 128
    assert n_bblk == nw
    eg = embed // nl

    mesh = plsc.VectorSubcoreMesh(core_axis_name="c", subcore_axis_name="s")

    @functools.partial(
        pl.kernel,
        mesh=mesh,
        out_type=jax.ShapeDtypeStruct((hist, embed // 8, n_bblk, 1024),
                                      jnp.float32),
        scratch_types=[
            pltpu.VMEM((8, 128), jnp.int32),
            [pltpu.VMEM((128, embed), jnp.float32) for _ in range(2)],
            [pltpu.VMEM((8 * 1024,), jnp.float32) for _ in range(2)],
            [pltpu.SemaphoreType.DMA for _ in range(2)],
            [pltpu.SemaphoreType.DMA for _ in range(2)],
        ],
        compiler_params=pltpu.CompilerParams(use_tc_tiling_on_sc=False,
                                             needs_layout_passes=False),
    )
    def k(idx_hbm, tbl_hbm, out_hbm, idx_v, rows_v, asm_v, sem_g, sem_o):
        w = lax.axis_index("s") * nc + lax.axis_index("c")
        row_iota = [lax.iota(jnp.int32, nl) + g * nl for g in range(128 // nl)]

        def do_hblk(hblk, carry):
            pltpu.sync_copy(idx_hbm.at[hblk, w], idx_v)
            pltpu.async_copy(tbl_hbm.at[idx_v.at[0]], rows_v[0], sem_g[0])

            for sh in range(8):
                b = sh % 2
                if sh < 7:
                    pltpu.async_copy(tbl_hbm.at[idx_v.at[sh + 1]],
                                     rows_v[(sh + 1) % 2], sem_g[(sh + 1) % 2])
                pltpu.make_async_copy(tbl_hbm.at[idx_v.at[sh]], rows_v[b],
                                      sem_g[b]).wait()

                @pl.when(jnp.logical_or(hblk > 0, sh >= 2))
                def _():
                    for t in range(embed // 8):
                        pltpu.make_async_copy(
                            asm_v[b].at[pl.ds(t * 1024, 1024)],
                            out_hbm.at[0, t, w], sem_o[b]).wait()

                def do_col(e, carry):
                    ce = jnp.full((nl,), e, jnp.int32)
                    for g in range(128 // nl):
                        val = plsc.load_gather(rows_v[b], [row_iota[g], ce])
                        asm_v[b][pl.ds(e * 128 + g * nl, nl)] = val
                    return carry

                lax.fori_loop(0, embed, do_col, 0, unroll=2)

                h = hblk * 8 + sh
                for t in range(embed // 8):
                    pltpu.async_copy(asm_v[b].at[pl.ds(t * 1024, 1024)],
                                     out_hbm.at[h, t, w], sem_o[b])
            return carry

        lax.fori_loop(0, n_hblk, do_hblk, 0)

        for b in range(2):
            for t in range(embed // 8):
                pltpu.make_async_copy(asm_v[b].at[pl.ds(t * 1024, 1024)],
                                      out_hbm.at[0, t, w], sem_o[b]).wait()

    return k


def kernel(input_tensor, embedding_table):
    batch, hist = input_tensor.shape
    vocab, embed = embedding_table.shape
    idx4 = (input_tensor.T.reshape(hist // 8, 8, batch // 128, 128)
            .transpose(0, 2, 1, 3))
    out5 = _build(batch, hist, vocab, embed)(idx4, embedding_table)
    out = (out5.reshape(hist, embed // 8, batch // 128, 8, 128)
           .transpose(2, 4, 0, 1, 3).reshape(batch, hist, embed))
    return out


# speedup vs baseline: 2.0249x; 2.0249x over previous
import functools

import jax
import jax.numpy as jnp
from jax import lax
from jax.experimental import pallas as pl
from jax.experimental.pallas import tpu as pltpu
from jax.experimental.pallas import tpu_sc as plsc


@functools.lru_cache(maxsize=None)
def _build_gather(b_total: int, embed: int):
    info = plsc.get_sparse_core_info()
    nc, ns = info.num_cores, info.num_subcores
    nw = nc * ns
    b_per_w = b_total // nw
    assert b_per_w * nw == b_total
    nbuf = 2
    chunk = 800
    while b_per_w % (chunk * nbuf) != 0:
        chunk //= 2
    n_groups = b_per_w // (chunk * nbuf)

    mesh = plsc.VectorSubcoreMesh(core_axis_name="c", subcore_axis_name="s")

    @functools.partial(
        pl.kernel,
        mesh=mesh,
        out_type=jax.ShapeDtypeStruct((b_total, 2 * embed), jnp.float32),
        scratch_types=[
            [pltpu.VMEM((chunk,), jnp.int32) for _ in range(nbuf)],
            [pltpu.VMEM((chunk, embed), jnp.float32) for _ in range(nbuf)],
            [pltpu.SemaphoreType.DMA for _ in range(nbuf)],
            [pltpu.SemaphoreType.DMA for _ in range(nbuf)],
        ],
        compiler_params=pltpu.CompilerParams(use_tc_tiling_on_sc=False,
                                             needs_layout_passes=False),
    )
    def gather_kernel(idx_hbm, table_hbm, out_hbm, idx_v, rows_v, sem_g, sem_o):
        wid = lax.axis_index("s") * nc + lax.axis_index("c")
        base0 = wid * b_per_w

        def out_slice(base):
            return out_hbm.at[pl.ds(base, chunk), pl.ds(0, embed)]

        for b in range(nbuf):
            base = base0 + b * chunk
            pltpu.sync_copy(idx_hbm.at[pl.ds(base, chunk)], idx_v[b])
            pltpu.async_copy(table_hbm.at[idx_v[b]], rows_v[b], sem_g[b])

        def body(g, carry):
            for b in range(nbuf):
                base = base0 + (g * nbuf + b) * chunk
                pltpu.make_async_copy(
                    table_hbm.at[idx_v[b]], rows_v[b], sem_g[b]).wait()
                pltpu.async_copy(rows_v[b], out_slice(base), sem_o[b])

                @pl.when(g < n_groups - 1)
                def _():
                    nbase = base + nbuf * chunk
                    pltpu.sync_copy(idx_hbm.at[pl.ds(nbase, chunk)], idx_v[b])
                    pltpu.make_async_copy(
                        rows_v[b], out_slice(base), sem_o[b]).wait()
                    pltpu.async_copy(table_hbm.at[idx_v[b]], rows_v[b],
                                     sem_g[b])

            return carry

        lax.fori_loop(0, n_groups, body, 0)

        for b in range(nbuf):
            base = base0 + b * chunk
            pltpu.make_async_copy(rows_v[b], out_slice(base), sem_o[b]).wait()

    return gather_kernel


def kernel(input_tensor, embedding_table):
    bsz, hist = input_tensor.shape
    embed = embedding_table.shape[1]
    idx = input_tensor.reshape(-1).astype(jnp.int32)
    out128 = _build_gather(bsz * hist, embed)(idx, embedding_table)
    return out128[:, :embed].reshape(bsz, hist, embed)
